# baseline (speedup 1.0000x reference)
import jax
import jax.numpy as jnp
from jax import lax
from jax.experimental import pallas as pl
from jax.experimental.pallas import tpu as pltpu
from jax.experimental.pallas import tpu_sc as plsc

N_NODES = 50000
D = 512
NSEG = 256
L = 16
NC = 2
NS = 16
NW = NC * NS
SPW = NSEG // NW
R = 40
NBUF = 3
HALF = 2
CH = D // HALF
CV = CH // L


def _pool_body(feat_hbm, ids_hbm, out_hbm, idsv, sp_ids, fbuf, ostage,
               sem0, sem1, sem2):
    wid = lax.axis_index("s") * NC + lax.axis_index("c")
    sbase = wid * SPW

    @pl.when(lax.axis_index("s") == 0)
    def _():
        pltpu.sync_copy(ids_hbm, sp_ids)

    plsc.subcore_barrier()
    pltpu.sync_copy(sp_ids, idsv)

    targets = sbase + lax.iota(jnp.int32, L)
    pos = jnp.zeros((L,), jnp.int32)
    step = 32768
    while step >= 1:
        npos = pos + step
        idx = jnp.minimum(npos - 1, N_NODES - 1)
        vals = plsc.load_gather(idsv, [idx])
        ok = (npos <= N_NODES) & (vals < targets)
        pos = jnp.where(ok, npos, pos)
        step //= 2

    s_bnds = [pos[k] for k in range(SPW + 1)]
    row_lo = s_bnds[0]
    row_hi = s_bnds[SPW]

    g_lo = row_lo // R
    nb = jnp.where(row_hi > row_lo, (row_hi + R - 1) // R - g_lo, 0)

    def bstart_of(t):
        return jnp.minimum((g_lo + t) * R, N_NODES - R)

    sems = (sem0, sem1, sem2)

    def copy_desc(t, b):
        return pltpu.make_async_copy(
            feat_hbm.at[pl.ds(bstart_of(t), R)], fbuf.at[b], sems[b])

    for b in range(NBUF):
        @pl.when(nb > b)
        def _(b=b):
            copy_desc(b, b).start()

    zeros = jnp.zeros((L,), jnp.float32)
    ninf = jnp.full((L,), -jnp.inf, jnp.float32)

    def init_body(k, c):
        for j in range(D // L):
            ostage[k, pl.ds(j * L, L)] = zeros
            ostage[k, pl.ds(D + j * L, L)] = ninf
        return c

    lax.fori_loop(0, SPW, init_body, 0)

    def process(t, b):
        g = (g_lo + t) * R
        proc_lo = jnp.maximum(row_lo, g)
        proc_hi = jnp.minimum(row_hi, g + R)
        bstart = bstart_of(t)
        buf = fbuf.at[b]

        for k in range(SPW):
            a = jnp.maximum(s_bnds[k], proc_lo)
            e = jnp.minimum(s_bnds[k + 1], proc_hi)

            @pl.when(e > a)
            def _():
                for h in range(HALF):
                    scol = h * CH
                    mcol = D + h * CH
                    carry0 = tuple(
                        ostage[k, pl.ds(scol + j * L, L)] for j in range(CV)
                    ) + tuple(
                        ostage[k, pl.ds(mcol + j * L, L)] for j in range(CV)
                    )

                    def row_body(r, carry):
                        ro = r - bstart
                        fs = [buf[ro, pl.ds(scol + j * L, L)]
                              for j in range(CV)]
                        sums = tuple(s + f for s, f in zip(carry[:CV], fs))
                        maxs = tuple(jnp.maximum(m, f)
                                     for m, f in zip(carry[CV:], fs))
                        return sums + maxs

                    carry = lax.fori_loop(a, e, row_body, carry0)
                    for j in range(CV):
                        ostage[k, pl.ds(scol + j * L, L)] = carry[j]
                        ostage[k, pl.ds(mcol + j * L, L)] = carry[CV + j]

    def ring_body(u, c):
        for b in range(NBUF):
            t = u * NBUF + b

            @pl.when(t < nb)
            def _(t=t, b=b):
                copy_desc(t, b).wait()
                process(t, b)

                @pl.when(t + NBUF < nb)
                def _():
                    copy_desc(t + NBUF, b).start()

        return c

    lax.fori_loop(0, (nb + NBUF - 1) // NBUF, ring_body, 0)

    pltpu.sync_copy(ostage, out_hbm.at[pl.ds(sbase, SPW)])


def kernel(feat, segment_ids):
    mesh = plsc.VectorSubcoreMesh(core_axis_name="c", subcore_axis_name="s")
    f = pl.kernel(
        _pool_body,
        out_type=jax.ShapeDtypeStruct((NSEG, 2 * D), jnp.float32),
        mesh=mesh,
        compiler_params=pltpu.CompilerParams(needs_layout_passes=False),
        scratch_types=[
            pltpu.VMEM((N_NODES,), jnp.int32),
            pltpu.VMEM_SHARED((N_NODES,), jnp.int32),
            pltpu.VMEM((NBUF, R, D), jnp.float32),
            pltpu.VMEM((SPW, 2 * D), jnp.float32),
            pltpu.SemaphoreType.DMA,
            pltpu.SemaphoreType.DMA,
            pltpu.SemaphoreType.DMA,
        ],
    )
    return f(feat, segment_ids)


# speedup vs baseline: 1.2308x; 1.2308x over previous
import jax
import jax.numpy as jnp
from jax import lax
from jax.experimental import pallas as pl
from jax.experimental.pallas import tpu as pltpu
from jax.experimental.pallas import tpu_sc as plsc

N_NODES = 50000
D = 512
NSEG = 256
L = 16
NC = 2
NS = 16
NW = NC * NS
SPW = NSEG // NW
R = 64
HALF = 2
CH = D // HALF
CV = CH // L


def _pool_body(feat_hbm, ids_hbm, out_hbm, idsv, sp_ids, fbuf, ostage,
               sem0, sem1):
    wid = lax.axis_index("s") * NC + lax.axis_index("c")
    sbase = wid * SPW

    @pl.when(lax.axis_index("s") == 0)
    def _():
        pltpu.sync_copy(ids_hbm, sp_ids)

    plsc.subcore_barrier()
    pltpu.sync_copy(sp_ids, idsv)

    targets = sbase + lax.iota(jnp.int32, L)
    pos = jnp.zeros((L,), jnp.int32)
    step = 32768
    while step >= 1:
        npos = pos + step
        idx = jnp.minimum(npos - 1, N_NODES - 1)
        vals = plsc.load_gather(idsv, [idx])
        ok = (npos <= N_NODES) & (vals < targets)
        pos = jnp.where(ok, npos, pos)
        step //= 2

    s_bnds = [pos[k] for k in range(SPW + 1)]
    row_lo = s_bnds[0]
    row_hi = s_bnds[SPW]

    g_lo = row_lo // R
    nb = jnp.where(row_hi > row_lo, (row_hi + R - 1) // R - g_lo, 0)

    def bstart_of(t):
        return jnp.minimum((g_lo + t) * R, N_NODES - R)

    def copy_desc(t, b):
        buf = fbuf.at[b]
        sem = sem0 if b == 0 else sem1
        return pltpu.make_async_copy(
            feat_hbm.at[pl.ds(bstart_of(t), R)], buf, sem)

    @pl.when(nb > 0)
    def _():
        copy_desc(0, 0).start()

    @pl.when(nb > 1)
    def _():
        copy_desc(1, 1).start()

    zeros = jnp.zeros((L,), jnp.float32)
    ninf = jnp.full((L,), -jnp.inf, jnp.float32)

    def init_body(k, c):
        for j in range(D // L):
            ostage[k, pl.ds(j * L, L)] = zeros
            ostage[k, pl.ds(D + j * L, L)] = ninf
        return c

    lax.fori_loop(0, SPW, init_body, 0)

    def process(t, b):
        g = (g_lo + t) * R
        proc_lo = jnp.maximum(row_lo, g)
        proc_hi = jnp.minimum(row_hi, g + R)
        bstart = bstart_of(t)
        buf = fbuf.at[b]

        for k in range(SPW):
            a = jnp.maximum(s_bnds[k], proc_lo)
            e = jnp.minimum(s_bnds[k + 1], proc_hi)

            @pl.when(e > a)
            def _():
                for h in range(HALF):
                    scol = h * CH
                    mcol = D + h * CH
                    carry0 = tuple(
                        ostage[k, pl.ds(scol + j * L, L)] for j in range(CV)
                    ) + tuple(
                        ostage[k, pl.ds(mcol + j * L, L)] for j in range(CV)
                    )

                    def row_body(r, carry):
                        ro = r - bstart
                        fs = [buf[ro, pl.ds(scol + j * L, L)]
                              for j in range(CV)]
                        sums = tuple(s + f for s, f in zip(carry[:CV], fs))
                        maxs = tuple(jnp.maximum(m, f)
                                     for m, f in zip(carry[CV:], fs))
                        return sums + maxs

                    carry = lax.fori_loop(a, e, row_body, carry0)
                    for j in range(CV):
                        ostage[k, pl.ds(scol + j * L, L)] = carry[j]
                        ostage[k, pl.ds(mcol + j * L, L)] = carry[CV + j]

    def pair_body(u, c):
        for b in range(2):
            t = u * 2 + b

            @pl.when(t < nb)
            def _():
                copy_desc(t, b).wait()
                process(t, b)

                @pl.when(t + 2 < nb)
                def _():
                    copy_desc(t + 2, b).start()

        return c

    lax.fori_loop(0, (nb + 1) // 2, pair_body, 0)

    pltpu.sync_copy(ostage, out_hbm.at[pl.ds(sbase, SPW)])


def kernel(feat, segment_ids):
    mesh = plsc.VectorSubcoreMesh(core_axis_name="c", subcore_axis_name="s")
    f = pl.kernel(
        _pool_body,
        out_type=jax.ShapeDtypeStruct((NSEG, 2 * D), jnp.float32),
        mesh=mesh,
        compiler_params=pltpu.CompilerParams(needs_layout_passes=False),
        scratch_types=[
            pltpu.VMEM((N_NODES,), jnp.int32),
            pltpu.VMEM_SHARED((N_NODES,), jnp.int32),
            pltpu.VMEM((2, R, D), jnp.float32),
            pltpu.VMEM((SPW, 2 * D), jnp.float32),
            pltpu.SemaphoreType.DMA,
            pltpu.SemaphoreType.DMA,
        ],
    )
    return f(feat, segment_ids)
